# baseline (speedup 1.0000x reference)
import functools

import jax
import jax.numpy as jnp
from jax import lax
from jax.experimental import pallas as pl
from jax.experimental.pallas import tpu as pltpu
from jax.experimental.pallas import tpu_sc as plsc

NUM_CORES = 2
NUM_SUBCORES = 16
NUM_WORKERS = NUM_CORES * NUM_SUBCORES

WT_BLOCK = 8192


def _wt_body(x_ref, o_ref):
    o_ref[:, 0:64] = x_ref[...].T


def _weight_to_rows(weight):
    vocab, dim = weight.shape
    wt = weight.T
    w128 = pl.pallas_call(
        _wt_body,
        out_shape=jax.ShapeDtypeStruct((vocab, 2 * dim), jnp.float32),
        grid=(pl.cdiv(vocab, WT_BLOCK),),
        in_specs=[pl.BlockSpec((dim, WT_BLOCK), lambda j: (0, j))],
        out_specs=pl.BlockSpec((WT_BLOCK, 2 * dim), lambda j: (j, 0)),
    )(wt)
    return w128.reshape(2 * vocab, dim)


def _transpose_block(rows_ref, tb_ref, gr_idx):
    @plsc.parallel_loop(0, 128, unroll=8)
    def _(c):
        c_vec = jnp.full((16,), c, jnp.int32)
        for k in range(4):
            g_idx, r_idx = gr_idx[k]
            v = rows_ref[c, pl.ds(16 * k, 16)]
            plsc.store_scatter(tb_ref, [g_idx, r_idx, c_vec], v)


def _gather_body(seq, weight_hbm, idx_hbm, out_hbm,
                 idx_v, rows0, rows1, tb0, tb1, sem0, sem1):
    wid = lax.axis_index("c") * NUM_SUBCORES + lax.axis_index("s")

    pltpu.sync_copy(idx_hbm.at[:, wid], idx_v)

    lane = lax.iota(jnp.int32, 16)
    gr_idx = [((16 * k + lane) >> 3, (16 * k + lane) & 7) for k in range(4)]

    c0 = pltpu.async_copy(weight_hbm.at[idx_v.at[0]], rows0, sem0)
    c1 = pltpu.async_copy(weight_hbm.at[idx_v.at[1]], rows1, sem1)
    del c0, c1

    @pl.loop(0, seq, step=2)
    def _(s):
        pltpu.make_async_copy(weight_hbm.at[idx_v.at[s]], rows0, sem0).wait()
        _transpose_block(rows0, tb0, gr_idx)

        @pl.when(s + 2 < seq)
        def _():
            pltpu.async_copy(weight_hbm.at[idx_v.at[s + 2]], rows0, sem0)

        pltpu.sync_copy(tb0.at[:, :, pl.ds(0, 128)], out_hbm.at[s, :, wid])

        pltpu.make_async_copy(weight_hbm.at[idx_v.at[s + 1]], rows1,
                              sem1).wait()
        _transpose_block(rows1, tb1, gr_idx)

        @pl.when(s + 3 < seq)
        def _():
            pltpu.async_copy(weight_hbm.at[idx_v.at[s + 3]], rows1, sem1)

        pltpu.sync_copy(tb1.at[:, :, pl.ds(0, 128)],
                        out_hbm.at[s + 1, :, wid])


def kernel(input_, weight):
    batch, seq = input_.shape
    vocab, dim = weight.shape
    assert batch == NUM_WORKERS * 128 and seq % 2 == 0 and dim == 64

    w2 = _weight_to_rows(weight)
    idx = (input_.astype(jnp.int32) * 2).T.reshape(seq, NUM_WORKERS, 128)

    mesh = plsc.VectorSubcoreMesh(core_axis_name="c", subcore_axis_name="s")
    sc_gather = pl.kernel(
        functools.partial(_gather_body, seq),
        out_type=jax.ShapeDtypeStruct((seq, 8, NUM_WORKERS, 8, 128),
                                      weight.dtype),
        mesh=mesh,
        scratch_types=[
            pltpu.VMEM((seq, 128), jnp.int32),
            pltpu.VMEM((128, dim), jnp.float32),
            pltpu.VMEM((128, dim), jnp.float32),
            pltpu.VMEM((8, 8, 129), jnp.float32),
            pltpu.VMEM((8, 8, 129), jnp.float32),
            pltpu.SemaphoreType.DMA,
            pltpu.SemaphoreType.DMA,
        ],
        compiler_params=pltpu.CompilerParams(use_tc_tiling_on_sc=False,
                                             needs_layout_passes=False),
    )
    x = sc_gather(w2, idx)
    return x.transpose(2, 4, 0, 1, 3).reshape(batch, seq, dim)


# speedup vs baseline: 1.0482x; 1.0482x over previous
import functools

import jax
import jax.numpy as jnp
from jax import lax
from jax.experimental import pallas as pl
from jax.experimental.pallas import tpu as pltpu
from jax.experimental.pallas import tpu_sc as plsc

NUM_CORES = 2
NUM_SUBCORES = 16
NUM_WORKERS = NUM_CORES * NUM_SUBCORES

WT_BLOCK = 16384


def _wt_body(x_ref, o_ref):
    o_ref[:, 0:64] = x_ref[...].T


def _weight_to_rows(weight):
    vocab, dim = weight.shape
    wt = weight.T
    w128 = pl.pallas_call(
        _wt_body,
        out_shape=jax.ShapeDtypeStruct((vocab, 2 * dim), jnp.float32),
        grid=(pl.cdiv(vocab, WT_BLOCK),),
        in_specs=[pl.BlockSpec((dim, WT_BLOCK), lambda j: (0, j))],
        out_specs=pl.BlockSpec((WT_BLOCK, 2 * dim), lambda j: (j, 0)),
    )(wt)
    return w128.reshape(2 * vocab, dim)


def _transpose_block(rows_ref, tb_ref, gr_idx):
    @plsc.parallel_loop(0, 128, unroll=8)
    def _(c):
        c_vec = jnp.full((16,), c, jnp.int32)
        for k in range(4):
            g_idx, r_idx = gr_idx[k]
            v = rows_ref[c, pl.ds(16 * k, 16)]
            plsc.store_scatter(tb_ref, [g_idx, r_idx, c_vec], v)


def _gather_body(seq, weight_hbm, idx_hbm, out_hbm,
                 idx_v, rows0, rows1, tb0, tb1, sem0, sem1):
    wid = lax.axis_index("c") * NUM_SUBCORES + lax.axis_index("s")

    pltpu.sync_copy(idx_hbm.at[:, wid], idx_v)

    lane = lax.iota(jnp.int32, 16)
    gr_idx = [((16 * k + lane) >> 3, (16 * k + lane) & 7) for k in range(4)]

    c0 = pltpu.async_copy(weight_hbm.at[idx_v.at[0]], rows0, sem0)
    c1 = pltpu.async_copy(weight_hbm.at[idx_v.at[1]], rows1, sem1)
    del c0, c1

    @pl.loop(0, seq, step=2)
    def _(s):
        pltpu.make_async_copy(weight_hbm.at[idx_v.at[s]], rows0, sem0).wait()
        _transpose_block(rows0, tb0, gr_idx)

        @pl.when(s + 2 < seq)
        def _():
            pltpu.async_copy(weight_hbm.at[idx_v.at[s + 2]], rows0, sem0)

        pltpu.sync_copy(tb0.at[:, :, pl.ds(0, 128)], out_hbm.at[s, :, wid])

        pltpu.make_async_copy(weight_hbm.at[idx_v.at[s + 1]], rows1,
                              sem1).wait()
        _transpose_block(rows1, tb1, gr_idx)

        @pl.when(s + 3 < seq)
        def _():
            pltpu.async_copy(weight_hbm.at[idx_v.at[s + 3]], rows1, sem1)

        pltpu.sync_copy(tb1.at[:, :, pl.ds(0, 128)],
                        out_hbm.at[s + 1, :, wid])


def kernel(input_, weight):
    batch, seq = input_.shape
    vocab, dim = weight.shape
    assert batch == NUM_WORKERS * 128 and seq % 2 == 0 and dim == 64

    w2 = _weight_to_rows(weight)
    idx = (input_.astype(jnp.int32) * 2).T.reshape(seq, NUM_WORKERS, 128)

    mesh = plsc.VectorSubcoreMesh(core_axis_name="c", subcore_axis_name="s")
    sc_gather = pl.kernel(
        functools.partial(_gather_body, seq),
        out_type=jax.ShapeDtypeStruct((seq, 8, NUM_WORKERS, 8, 128),
                                      weight.dtype),
        mesh=mesh,
        scratch_types=[
            pltpu.VMEM((seq, 128), jnp.int32),
            pltpu.VMEM((128, dim), jnp.float32),
            pltpu.VMEM((128, dim), jnp.float32),
            pltpu.VMEM((8, 8, 129), jnp.float32),
            pltpu.VMEM((8, 8, 129), jnp.float32),
            pltpu.SemaphoreType.DMA,
            pltpu.SemaphoreType.DMA,
        ],
        compiler_params=pltpu.CompilerParams(use_tc_tiling_on_sc=False,
                                             needs_layout_passes=False),
    )
    x = sc_gather(w2, idx)
    return x.transpose(2, 4, 0, 1, 3).reshape(batch, seq, dim)


# speedup vs baseline: 1.1081x; 1.0571x over previous
import functools

import jax
import jax.numpy as jnp
from jax import lax
from jax.experimental import pallas as pl
from jax.experimental.pallas import tpu as pltpu
from jax.experimental.pallas import tpu_sc as plsc

NUM_CORES = 2
NUM_SUBCORES = 16
NUM_WORKERS = NUM_CORES * NUM_SUBCORES

WT_BLOCK = 16384


def _wt_body(x_ref, o_ref):
    o_ref[:, 0:64] = x_ref[...].T


def _weight_to_rows(weight):
    vocab, dim = weight.shape
    wt = weight.T
    w128 = pl.pallas_call(
        _wt_body,
        out_shape=jax.ShapeDtypeStruct((vocab, 2 * dim), jnp.float32),
        grid=(pl.cdiv(vocab, WT_BLOCK),),
        in_specs=[pl.BlockSpec((dim, WT_BLOCK), lambda j: (0, j))],
        out_specs=pl.BlockSpec((WT_BLOCK, 2 * dim), lambda j: (j, 0)),
    )(wt)
    return w128.reshape(2 * vocab, dim)


def _transpose_block(rows_ref, tb_ref, gr_idx):
    @plsc.parallel_loop(0, 128, unroll=8)
    def _(c):
        c_vec = jnp.full((16,), c, jnp.int32)
        for k in range(4):
            g_idx, r_idx = gr_idx[k]
            v = rows_ref[c, pl.ds(16 * k, 16)]
            plsc.store_scatter(tb_ref, [g_idx, r_idx, c_vec], v)


NBUF = 4


def _gather_body(seq, weight_hbm, idx_hbm, out_hbm, idx_v, *bufs):
    rows = bufs[0:NBUF]
    tbs = bufs[NBUF:2 * NBUF]
    gsems = bufs[2 * NBUF:3 * NBUF]
    wsems = bufs[3 * NBUF:4 * NBUF]
    wid = lax.axis_index("c") * NUM_SUBCORES + lax.axis_index("s")

    pltpu.sync_copy(idx_hbm.at[:, wid], idx_v)

    lane = lax.iota(jnp.int32, 16)
    gr_idx = [((16 * k + lane) >> 3, (16 * k + lane) & 7) for k in range(4)]

    for u in range(NBUF):
        pltpu.async_copy(weight_hbm.at[idx_v.at[u]], rows[u], gsems[u])

    @pl.loop(0, seq, step=NBUF)
    def _(s):
        for u in range(NBUF):
            pltpu.make_async_copy(weight_hbm.at[idx_v.at[s + u]], rows[u],
                                  gsems[u]).wait()

            @pl.when(s > 0)
            def _():
                pltpu.make_async_copy(tbs[u].at[:, :, pl.ds(0, 128)],
                                      out_hbm.at[s, :, wid], wsems[u]).wait()

            _transpose_block(rows[u], tbs[u], gr_idx)

            @pl.when(s + u + NBUF < seq)
            def _():
                pltpu.async_copy(weight_hbm.at[idx_v.at[s + u + NBUF]],
                                 rows[u], gsems[u])

            pltpu.async_copy(tbs[u].at[:, :, pl.ds(0, 128)],
                             out_hbm.at[s + u, :, wid], wsems[u])

    for u in range(NBUF):
        pltpu.make_async_copy(tbs[u].at[:, :, pl.ds(0, 128)],
                              out_hbm.at[0, :, wid], wsems[u]).wait()


def kernel(input_, weight):
    batch, seq = input_.shape
    vocab, dim = weight.shape
    assert batch == NUM_WORKERS * 128 and seq % 2 == 0 and dim == 64

    w2 = _weight_to_rows(weight)
    idx = (input_.astype(jnp.int32) * 2).T.reshape(seq, NUM_WORKERS, 128)

    mesh = plsc.VectorSubcoreMesh(core_axis_name="c", subcore_axis_name="s")
    sc_gather = pl.kernel(
        functools.partial(_gather_body, seq),
        out_type=jax.ShapeDtypeStruct((seq, 8, NUM_WORKERS, 8, 128),
                                      weight.dtype),
        mesh=mesh,
        scratch_types=(
            [pltpu.VMEM((seq, 128), jnp.int32)]
            + [pltpu.VMEM((128, dim), jnp.float32)] * NBUF
            + [pltpu.VMEM((8, 8, 129), jnp.float32)] * NBUF
            + [pltpu.SemaphoreType.DMA] * (2 * NBUF)
        ),
        compiler_params=pltpu.CompilerParams(use_tc_tiling_on_sc=False,
                                             needs_layout_passes=False),
    )
    x = sc_gather(w2, idx)
    return x.transpose(2, 4, 0, 1, 3).reshape(batch, seq, dim)


# speedup vs baseline: 1.1449x; 1.0332x over previous
import functools

import jax
import jax.numpy as jnp
from jax import lax
from jax.experimental import pallas as pl
from jax.experimental.pallas import tpu as pltpu
from jax.experimental.pallas import tpu_sc as plsc

NUM_CORES = 2
NUM_SUBCORES = 16
NUM_WORKERS = NUM_CORES * NUM_SUBCORES

WT_BLOCK = 8192
PAIR_OFF = 524288


def _wt_body(x1_ref, x2_ref, o_ref):
    o_ref[:, 0:64] = x1_ref[...].T
    o_ref[:, 64:128] = x2_ref[...].T


def _weight_to_rows(weight):
    vocab, dim = weight.shape
    wt = weight.T
    k_blocks = PAIR_OFF // WT_BLOCK
    n_blocks = pl.cdiv(vocab, WT_BLOCK)
    w128 = pl.pallas_call(
        _wt_body,
        out_shape=jax.ShapeDtypeStruct((PAIR_OFF, 2 * dim), jnp.float32),
        grid=(k_blocks,),
        in_specs=[
            pl.BlockSpec((dim, WT_BLOCK), lambda j: (0, j)),
            pl.BlockSpec((dim, WT_BLOCK),
                         lambda j: (0, jnp.minimum(j + k_blocks,
                                                   n_blocks - 1))),
        ],
        out_specs=pl.BlockSpec((WT_BLOCK, 2 * dim), lambda j: (j, 0)),
    )(wt, wt)
    return w128.reshape(2 * PAIR_OFF, dim)


def _transpose_block(rows_ref, tb_ref, gr_idx):
    @plsc.parallel_loop(0, 128, unroll=8)
    def _(c):
        c_vec = jnp.full((16,), c, jnp.int32)
        for k in range(4):
            g_idx, r_idx = gr_idx[k]
            v = rows_ref[c, pl.ds(16 * k, 16)]
            plsc.store_scatter(tb_ref, [g_idx, r_idx, c_vec], v)


NBUF = 4


def _gather_body(seq, weight_hbm, idx_hbm, out_hbm, idx_v, *bufs):
    rows = bufs[0:NBUF]
    tbs = bufs[NBUF:2 * NBUF]
    gsems = bufs[2 * NBUF:3 * NBUF]
    wsems = bufs[3 * NBUF:4 * NBUF]
    wid = lax.axis_index("c") * NUM_SUBCORES + lax.axis_index("s")

    pltpu.sync_copy(idx_hbm.at[:, wid], idx_v)

    lane = lax.iota(jnp.int32, 16)
    gr_idx = [((16 * k + lane) >> 3, (16 * k + lane) & 7) for k in range(4)]

    for u in range(NBUF):
        pltpu.async_copy(weight_hbm.at[idx_v.at[u]], rows[u], gsems[u])

    @pl.loop(0, seq, step=NBUF)
    def _(s):
        for u in range(NBUF):
            pltpu.make_async_copy(weight_hbm.at[idx_v.at[s + u]], rows[u],
                                  gsems[u]).wait()

            @pl.when(s > 0)
            def _():
                pltpu.make_async_copy(tbs[u].at[:, :, pl.ds(0, 128)],
                                      out_hbm.at[s, :, wid], wsems[u]).wait()

            _transpose_block(rows[u], tbs[u], gr_idx)

            @pl.when(s + u + NBUF < seq)
            def _():
                pltpu.async_copy(weight_hbm.at[idx_v.at[s + u + NBUF]],
                                 rows[u], gsems[u])

            pltpu.async_copy(tbs[u].at[:, :, pl.ds(0, 128)],
                             out_hbm.at[s + u, :, wid], wsems[u])

    for u in range(NBUF):
        pltpu.make_async_copy(tbs[u].at[:, :, pl.ds(0, 128)],
                              out_hbm.at[0, :, wid], wsems[u]).wait()


def kernel(input_, weight):
    batch, seq = input_.shape
    vocab, dim = weight.shape
    assert batch == NUM_WORKERS * 128 and seq % 2 == 0 and dim == 64

    w2 = _weight_to_rows(weight)
    v = input_.astype(jnp.int32)
    idx = (2 * v - jnp.where(v < PAIR_OFF, 0, 2 * PAIR_OFF - 1)
           ).T.reshape(seq, NUM_WORKERS, 128)

    mesh = plsc.VectorSubcoreMesh(core_axis_name="c", subcore_axis_name="s")
    sc_gather = pl.kernel(
        functools.partial(_gather_body, seq),
        out_type=jax.ShapeDtypeStruct((seq, 8, NUM_WORKERS, 8, 128),
                                      weight.dtype),
        mesh=mesh,
        scratch_types=(
            [pltpu.VMEM((seq, 128), jnp.int32)]
            + [pltpu.VMEM((128, dim), jnp.float32)] * NBUF
            + [pltpu.VMEM((8, 8, 129), jnp.float32)] * NBUF
            + [pltpu.SemaphoreType.DMA] * (2 * NBUF)
        ),
        compiler_params=pltpu.CompilerParams(use_tc_tiling_on_sc=False,
                                             needs_layout_passes=False),
    )
    x = sc_gather(w2, idx)
    return x.transpose(2, 4, 0, 1, 3).reshape(batch, seq, dim)


# speedup vs baseline: 1.1910x; 1.0403x over previous
import functools

import jax
import jax.numpy as jnp
from jax import lax
from jax.experimental import pallas as pl
from jax.experimental.pallas import tpu as pltpu
from jax.experimental.pallas import tpu_sc as plsc

NUM_CORES = 2
NUM_SUBCORES = 16
NUM_WORKERS = NUM_CORES * NUM_SUBCORES

WT_BLOCK = 16384
PAIR_OFF = 524288


def _wt_body(x1_ref, x2_ref, o_ref):
    o_ref[:, 0:64] = x1_ref[...].T
    o_ref[:, 64:128] = x2_ref[...].T


def _weight_to_rows(weight):
    vocab, dim = weight.shape
    wt = weight.T
    k_blocks = PAIR_OFF // WT_BLOCK
    n_blocks = pl.cdiv(vocab, WT_BLOCK)
    w128 = pl.pallas_call(
        _wt_body,
        out_shape=jax.ShapeDtypeStruct((PAIR_OFF, 2 * dim), jnp.float32),
        grid=(k_blocks,),
        in_specs=[
            pl.BlockSpec((dim, WT_BLOCK), lambda j: (0, j)),
            pl.BlockSpec((dim, WT_BLOCK),
                         lambda j: (0, jnp.minimum(j + k_blocks,
                                                   n_blocks - 1))),
        ],
        out_specs=pl.BlockSpec((WT_BLOCK, 2 * dim), lambda j: (j, 0)),
    )(wt, wt)
    return w128.reshape(2 * PAIR_OFF, dim)


def _transpose_block(rows_ref, tb_ref, gr_idx):
    @plsc.parallel_loop(0, 128, unroll=8)
    def _(c):
        c_vec = jnp.full((16,), c, jnp.int32)
        for k in range(4):
            g_idx, r_idx = gr_idx[k]
            v = rows_ref[c, pl.ds(16 * k, 16)]
            plsc.store_scatter(tb_ref, [g_idx, r_idx, c_vec], v)


NBUF = 4


def _gather_body(seq, weight_hbm, idx_hbm, out_hbm, idx_v, *bufs):
    rows = bufs[0:NBUF]
    tbs = bufs[NBUF:2 * NBUF]
    gsems = bufs[2 * NBUF:3 * NBUF]
    wsems = bufs[3 * NBUF:4 * NBUF]
    wid = lax.axis_index("c") * NUM_SUBCORES + lax.axis_index("s")

    pltpu.sync_copy(idx_hbm.at[:, wid], idx_v)

    lane = lax.iota(jnp.int32, 16)
    gr_idx = [((16 * k + lane) >> 3, (16 * k + lane) & 7) for k in range(4)]

    for u in range(NBUF):
        pltpu.async_copy(weight_hbm.at[idx_v.at[u]], rows[u], gsems[u])

    @pl.loop(0, seq, step=NBUF)
    def _(s):
        for u in range(NBUF):
            pltpu.make_async_copy(weight_hbm.at[idx_v.at[s + u]], rows[u],
                                  gsems[u]).wait()

            @pl.when(s > 0)
            def _():
                pltpu.make_async_copy(tbs[u].at[:, :, pl.ds(0, 128)],
                                      out_hbm.at[s, :, wid], wsems[u]).wait()

            _transpose_block(rows[u], tbs[u], gr_idx)

            @pl.when(s + u + NBUF < seq)
            def _():
                pltpu.async_copy(weight_hbm.at[idx_v.at[s + u + NBUF]],
                                 rows[u], gsems[u])

            pltpu.async_copy(tbs[u].at[:, :, pl.ds(0, 128)],
                             out_hbm.at[s + u, :, wid], wsems[u])

    for u in range(NBUF):
        pltpu.make_async_copy(tbs[u].at[:, :, pl.ds(0, 128)],
                              out_hbm.at[0, :, wid], wsems[u]).wait()


def kernel(input_, weight):
    batch, seq = input_.shape
    vocab, dim = weight.shape
    assert batch == NUM_WORKERS * 128 and seq % 2 == 0 and dim == 64

    w2 = _weight_to_rows(weight)
    v = input_.astype(jnp.int32)
    idx = (2 * v - jnp.where(v < PAIR_OFF, 0, 2 * PAIR_OFF - 1)
           ).T.reshape(seq, NUM_WORKERS, 128)

    mesh = plsc.VectorSubcoreMesh(core_axis_name="c", subcore_axis_name="s")
    sc_gather = pl.kernel(
        functools.partial(_gather_body, seq),
        out_type=jax.ShapeDtypeStruct((seq, 8, NUM_WORKERS, 8, 128),
                                      weight.dtype),
        mesh=mesh,
        scratch_types=(
            [pltpu.VMEM((seq, 128), jnp.int32)]
            + [pltpu.VMEM((128, dim), jnp.float32)] * NBUF
            + [pltpu.VMEM((8, 8, 129), jnp.float32)] * NBUF
            + [pltpu.SemaphoreType.DMA] * (2 * NBUF)
        ),
        compiler_params=pltpu.CompilerParams(use_tc_tiling_on_sc=False,
                                             needs_layout_passes=False),
    )
    x = sc_gather(w2, idx)
    return x.transpose(2, 4, 0, 1, 3).reshape(batch, seq, dim)
